# baseline (speedup 1.0000x reference)
import functools

import jax
import jax.numpy as jnp
from jax import lax
from jax.experimental import pallas as pl
from jax.experimental.pallas import tpu as pltpu
from jax.experimental.pallas import tpu_sc as plsc

B_ = 8
N_ = 512
H_ = 16
T_ = B_ * N_ * N_
MAXD = 510
ZROW = MAXD + 2

_INFO = plsc.get_sparse_core_info()
NC = _INFO.num_cores
NS = _INFO.num_subcores
NW = NC * NS
PER_W = T_ // NW

SUB = 128
NSUB = 16
CHUNK = SUB * NSUB
NCHUNK = PER_W // CHUNK
LANES = 16


NPLANE = CHUNK // N_


def _body(idx_hbm, tab_hbm, out_hbm, idx_v, rows_v, sem):
    wid = lax.axis_index("s") * NC + lax.axis_index("c")
    row0 = wid * (PER_W // SUB)

    def chunk(k, carry):
        r0 = row0 + k * NSUB
        pltpu.sync_copy(idx_hbm.at[pl.ds(r0, NSUB), :], idx_v)
        for j in range(NSUB):
            for i in range(SUB // LANES):
                iv = idx_v[j, pl.ds(i * LANES, LANES)]
                cl = jnp.minimum(jnp.maximum(iv, 0), MAXD + 1)
                idx_v[j, pl.ds(i * LANES, LANES)] = jnp.where(iv < 0, ZROW, cl)
        cps = []
        for j in range(NSUB):
            cps.append(pltpu.async_copy(
                tab_hbm.at[idx_v.at[j]],
                rows_v.at[j // NPLANE, pl.ds((j % NPLANE) * SUB, SUB), :],
                sem))
        for cp in cps:
            cp.wait()
        pltpu.sync_copy(
            rows_v, out_hbm.at[pl.ds(r0 * SUB // N_, NPLANE), :, :])
        return carry

    lax.fori_loop(0, NCHUNK, chunk, 0)


def kernel(raw_inputs, embedding_table):
    idx2d = raw_inputs.astype(jnp.int32).reshape(T_ // SUB, SUB)
    tab_aug = jnp.concatenate(
        [embedding_table, jnp.zeros((1, H_), jnp.float32)], axis=0)
    mesh = plsc.VectorSubcoreMesh(core_axis_name="c", subcore_axis_name="s")
    run = functools.partial(
        pl.kernel,
        mesh=mesh,
        out_type=jax.ShapeDtypeStruct((B_ * N_, N_, H_), jnp.float32),
        scratch_types=[
            pltpu.VMEM((NSUB, SUB), jnp.int32),
            pltpu.VMEM((NPLANE, N_, H_), jnp.float32),
            pltpu.SemaphoreType.DMA,
        ],
        compiler_params=pltpu.CompilerParams(use_tc_tiling_on_sc=False),
    )(_body)
    out = run(idx2d, tab_aug)
    return out.reshape(B_, N_, N_, H_)


# speedup vs baseline: 1.3306x; 1.3306x over previous
import functools

import jax
import jax.numpy as jnp
from jax import lax
from jax.experimental import pallas as pl
from jax.experimental.pallas import tpu as pltpu
from jax.experimental.pallas import tpu_sc as plsc

B_ = 8
N_ = 512
H_ = 16
NPLANES = B_ * N_
MAXD = 510
ZROW = MAXD + 2
TABW = (ZROW + 1) * H_

_INFO = plsc.get_sparse_core_info()
NC = _INFO.num_cores
NS = _INFO.num_subcores
NW = NC * NS
PW = NPLANES // NW
PC = 4
NCHUNK = PW // PC
LANES = 16
NGRP = N_ // LANES


def _body(idx_hbm, tab_hbm, out_hbm, tab_v, idx_v, out_v, sem):
    wid = lax.axis_index("s") * NC + lax.axis_index("c")
    p0 = wid * PW
    pltpu.sync_copy(tab_hbm, tab_v)

    def chunk(k, carry):
        pp = p0 + k * PC
        pltpu.sync_copy(idx_hbm.at[pl.ds(pp, PC), :], idx_v)
        for p in range(PC):
            def grp(g, c2):
                iv = idx_v[p, pl.ds(g * LANES, LANES)]
                cl = jnp.minimum(jnp.maximum(iv, 0), MAXD + 1)
                fl = jnp.where(iv < 0, ZROW, cl) * H_
                for h in range(H_):
                    out_v[p, h, pl.ds(g * LANES, LANES)] = (
                        plsc.load_gather(tab_v, [fl + h]))
                return c2
            lax.fori_loop(0, NGRP, grp, 0)
        pltpu.sync_copy(out_v, out_hbm.at[pl.ds(pp, PC), :, :])
        return carry

    lax.fori_loop(0, NCHUNK, chunk, 0)


def kernel(raw_inputs, embedding_table):
    idx2d = raw_inputs.astype(jnp.int32).reshape(NPLANES, N_)
    tab_flat = jnp.concatenate(
        [embedding_table, jnp.zeros((1, H_), jnp.float32)], axis=0).reshape(-1)
    mesh = plsc.VectorSubcoreMesh(core_axis_name="c", subcore_axis_name="s")
    run = functools.partial(
        pl.kernel,
        mesh=mesh,
        out_type=jax.ShapeDtypeStruct((NPLANES, H_, N_), jnp.float32),
        scratch_types=[
            pltpu.VMEM((TABW,), jnp.float32),
            pltpu.VMEM((PC, N_), jnp.int32),
            pltpu.VMEM((PC, H_, N_), jnp.float32),
            pltpu.SemaphoreType.DMA,
        ],
        compiler_params=pltpu.CompilerParams(
            use_tc_tiling_on_sc=False, needs_layout_passes=False),
    )(_body)
    out = run(idx2d, tab_flat)
    return out.reshape(B_, N_, H_, N_).transpose(0, 1, 3, 2)


# speedup vs baseline: 1.9788x; 1.4871x over previous
import functools

import jax
import jax.numpy as jnp
from jax import lax
from jax.experimental import pallas as pl
from jax.experimental.pallas import tpu as pltpu
from jax.experimental.pallas import tpu_sc as plsc

B_ = 8
N_ = 512
H_ = 16
NPLANES = B_ * N_
MAXD = 510
ZROW = MAXD + 2
TABW = (ZROW + 1) * H_

_INFO = plsc.get_sparse_core_info()
NC = _INFO.num_cores
NS = _INFO.num_subcores
NW = NC * NS
PW = NPLANES // NW
PC = 4
NCHUNK = PW // PC
LANES = 16
NGRP = N_ // LANES


NPAD = N_ + 1


def _body(idx_hbm, tab_hbm, out_hbm, tab_v, idx_v, out_v, sem):
    wid = lax.axis_index("s") * NC + lax.axis_index("c")
    p0 = wid * PW
    pltpu.sync_copy(tab_hbm, tab_v)
    iota = lax.iota(jnp.int32, LANES)

    def chunk(k, carry):
        pp = p0 + k * PC
        pltpu.sync_copy(idx_hbm.at[pl.ds(pp, PC), :], idx_v)
        for p in range(PC):
            def xform(g, c2):
                iv = idx_v[p, pl.ds(g * LANES, LANES)]
                cl = jnp.minimum(jnp.maximum(iv, 0), MAXD + 1)
                idx_v[p, pl.ds(g * LANES, LANES)] = (
                    jnp.where(iv < 0, ZROW, cl) * H_)
                return c2
            lax.fori_loop(0, NGRP, xform, 0)
        for p in range(PC):
            def grp(g, c2):
                j0 = g * LANES
                iv = idx_v[p, pl.ds(j0, LANES)]
                for u in range(LANES):
                    fj = iv[u]
                    row = plsc.load_gather(tab_v, [fj + iota])
                    plsc.store_scatter(
                        out_v, [jnp.zeros((LANES,), jnp.int32) + p, iota,
                                jnp.zeros((LANES,), jnp.int32) + (j0 + u)],
                        row)
                return c2
            lax.fori_loop(0, NGRP, grp, 0)
        pltpu.sync_copy(out_v.at[:, :, pl.ds(0, N_)],
                        out_hbm.at[pl.ds(pp, PC), :, :])
        return carry

    lax.fori_loop(0, NCHUNK, chunk, 0)


def kernel(raw_inputs, embedding_table):
    idx2d = raw_inputs.astype(jnp.int32).reshape(NPLANES, N_)
    tab_flat = jnp.concatenate(
        [embedding_table, jnp.zeros((1, H_), jnp.float32)], axis=0).reshape(-1)
    mesh = plsc.VectorSubcoreMesh(core_axis_name="c", subcore_axis_name="s")
    run = functools.partial(
        pl.kernel,
        mesh=mesh,
        out_type=jax.ShapeDtypeStruct((NPLANES, H_, N_), jnp.float32),
        scratch_types=[
            pltpu.VMEM((TABW,), jnp.float32),
            pltpu.VMEM((PC, N_), jnp.int32),
            pltpu.VMEM((PC, H_, NPAD), jnp.float32),
            pltpu.SemaphoreType.DMA,
        ],
        compiler_params=pltpu.CompilerParams(
            use_tc_tiling_on_sc=False, needs_layout_passes=False),
    )(_body)
    out = run(idx2d, tab_flat)
    return out.reshape(B_, N_, H_, N_).transpose(0, 1, 3, 2)


# speedup vs baseline: 2.2849x; 1.1547x over previous
import functools

import jax
import jax.numpy as jnp
from jax import lax
from jax.experimental import pallas as pl
from jax.experimental.pallas import tpu as pltpu
from jax.experimental.pallas import tpu_sc as plsc

B_ = 8
N_ = 512
H_ = 16
NPLANES = B_ * N_
MAXD = 510
ZROW = MAXD + 2
TABW = (ZROW + 1) * H_

_INFO = plsc.get_sparse_core_info()
NC = _INFO.num_cores
NS = _INFO.num_subcores
NW = NC * NS
PW = NPLANES // NW
PC = 4
NCHUNK = PW // PC
LANES = 16
NGRP = N_ // LANES
NPAD = N_ + 1


def _compute_chunk(tab_v, idx_v, out_v, iota):
    def xform(g, c2):
        iv = idx_v[pl.ds(g * LANES, LANES)]
        cl = jnp.minimum(jnp.maximum(iv, 0), MAXD + 1)
        idx_v[pl.ds(g * LANES, LANES)] = jnp.where(iv < 0, ZROW, cl) * H_
        return c2
    lax.fori_loop(0, PC * NGRP, xform, 0, unroll=4)

    zeros = jnp.zeros((LANES,), jnp.int32)
    for p in range(PC):
        plane = out_v.at[p]

        def grp(g, c2):
            j0 = g * LANES
            iv = idx_v[pl.ds(p * N_ + j0, LANES)]
            jb = zeros + j0
            for u in range(LANES):
                row = plsc.load_gather(tab_v, [iv[u] + iota])
                plsc.store_scatter(plane, [iota, jb + u], row)
            return c2
        lax.fori_loop(0, NGRP, grp, 0)


def _body(idx_hbm, tab_hbm, out_hbm, tab_v, idx_v, out_v, sem_t, sem_i,
          sem_o):
    wid = lax.axis_index("s") * NC + lax.axis_index("c")
    p0 = wid * PW
    pltpu.async_copy(tab_hbm, tab_v, sem_t).wait()
    iota = lax.iota(jnp.int32, LANES)

    def in_copy(k, b):
        off = (p0 + jnp.minimum(k, NCHUNK - 1) * PC) * N_
        return pltpu.async_copy(
            idx_hbm.at[pl.ds(off, PC * N_)], idx_v.at[b], sem_i)

    def out_copy(k, b):
        return pltpu.async_copy(
            out_v.at[b, :, :, pl.ds(0, N_)],
            out_hbm.at[pl.ds(p0 + k * PC, PC), :, :], sem_o)

    def wait_in(b):
        pltpu.make_async_copy(
            idx_hbm.at[pl.ds(0, PC * N_)], idx_v.at[b], sem_i).wait()

    def wait_out(b):
        pltpu.make_async_copy(
            out_v.at[b, :, :, pl.ds(0, N_)],
            out_hbm.at[pl.ds(p0, PC), :, :], sem_o).wait()

    in_copy(0, 0)
    in_copy(1, 1)
    for b in (0, 1):
        wait_in(b)
        _compute_chunk(tab_v, idx_v.at[b], out_v.at[b], iota)
        out_copy(b, b)
        in_copy(b + 2, b)

    def pair(k2, carry):
        for b in (0, 1):
            k = k2 * 2 + b
            wait_in(b)
            wait_out(b)
            _compute_chunk(tab_v, idx_v.at[b], out_v.at[b], iota)
            out_copy(k, b)
            in_copy(k + 2, b)
        return carry

    lax.fori_loop(1, NCHUNK // 2, pair, 0)
    for b in (0, 1):
        wait_in(b)
        wait_out(b)


def kernel(raw_inputs, embedding_table):
    idx1d = raw_inputs.astype(jnp.int32).reshape(NPLANES * N_)
    tab_flat = jnp.concatenate(
        [embedding_table, jnp.zeros((1, H_), jnp.float32)], axis=0).reshape(-1)
    mesh = plsc.VectorSubcoreMesh(core_axis_name="c", subcore_axis_name="s")
    run = functools.partial(
        pl.kernel,
        mesh=mesh,
        out_type=jax.ShapeDtypeStruct((NPLANES, H_, N_), jnp.float32),
        scratch_types=[
            pltpu.VMEM((TABW,), jnp.float32),
            pltpu.VMEM((2, PC * N_), jnp.int32),
            pltpu.VMEM((2, PC, H_, NPAD), jnp.float32),
            pltpu.SemaphoreType.DMA,
            pltpu.SemaphoreType.DMA,
            pltpu.SemaphoreType.DMA,
        ],
        compiler_params=pltpu.CompilerParams(
            use_tc_tiling_on_sc=False, needs_layout_passes=False),
    )(_body)
    out = run(idx1d, tab_flat)
    return out.reshape(B_, N_, H_, N_).transpose(0, 1, 3, 2)
